# baseline (speedup 1.0000x reference)
import functools

import jax
import jax.numpy as jnp
from jax.experimental import pallas as pl
from jax.experimental.pallas import tpu as pltpu

T, H, V = 2048, 2048, 32000
NH, NKV, DH = 16, 4, 128
E, K, I = 16, 2, 1024
EPS = 1e-6
THETA = 1000000.0
F32 = jnp.float32


def _rms(x, w_row):
    v = jnp.mean(x * x, axis=-1, keepdims=True)
    return x * jax.lax.rsqrt(v + EPS) * (1.0 + w_row)


BF16 = jnp.bfloat16


_BT1 = 128


def _prefc_body(ids_ref, emb_hbm, hid_ref, fc_ref, wemb_ref, whid_ref,
                win_ref, resid_ref, xn_ref, esc, sem):
    i = pl.program_id(0)
    base = i * _BT1
    copies = []
    for j in range(_BT1):
        idx = ids_ref[base + j]
        cp = pltpu.make_async_copy(
            emb_hbm.at[pl.ds(idx, 1), :], esc.at[pl.ds(j, 1), :], sem)
        cp.start()
        copies.append(cp)
    for cp in copies:
        cp.wait()
    emb_n = _rms(esc[...], wemb_ref[...])
    hid_n = _rms(hid_ref[...], whid_ref[...])
    x = (jnp.dot(emb_n, fc_ref[:H, :], preferred_element_type=F32)
         + jnp.dot(hid_n, fc_ref[H:, :], preferred_element_type=F32))
    resid_ref[...] = x
    xn_ref[...] = _rms(x, win_ref[...])


def _prefc(input_ids, hidden_states, fc_w, wemb, whid, win, embed_w):
    grid = (T // _BT1,)
    return pl.pallas_call(
        _prefc_body,
        grid_spec=pltpu.PrefetchScalarGridSpec(
            num_scalar_prefetch=1,
            grid=grid,
            in_specs=[
                pl.BlockSpec(memory_space=pl.ANY),
                pl.BlockSpec((_BT1, H), lambda i, ids: (i, 0)),
                pl.BlockSpec((2 * H, H), lambda i, ids: (0, 0)),
                pl.BlockSpec((1, H), lambda i, ids: (0, 0)),
                pl.BlockSpec((1, H), lambda i, ids: (0, 0)),
                pl.BlockSpec((1, H), lambda i, ids: (0, 0)),
            ],
            out_specs=[
                pl.BlockSpec((_BT1, H), lambda i, ids: (i, 0)),
                pl.BlockSpec((_BT1, H), lambda i, ids: (i, 0)),
            ],
            scratch_shapes=[
                pltpu.VMEM((_BT1, H), F32),
                pltpu.SemaphoreType.DMA,
            ],
        ),
        out_shape=[
            jax.ShapeDtypeStruct((T, H), F32),
            jax.ShapeDtypeStruct((T, H), F32),
        ],
        compiler_params=pltpu.CompilerParams(
            dimension_semantics=("parallel",),
            vmem_limit_bytes=100 * 1024 * 1024,
        ),
    )(input_ids, embed_w, hidden_states, fc_w, wemb, whid, win)


_BT2 = 256


def _qkv_body(xn_ref, wq_ref, wk_ref, wv_ref, qw_ref, kw_ref,
              cos_ref, sin_ref, q_ref, k_ref, v_ref):
    xn = xn_ref[...]
    q = jnp.dot(xn, wq_ref[...], preferred_element_type=F32)
    k = jnp.dot(xn, wk_ref[...], preferred_element_type=F32)
    v_ref[...] = jnp.dot(xn, wv_ref[...], preferred_element_type=F32)
    c, s = cos_ref[...], sin_ref[...]

    def norm_rope(x, h, w_row):
        xh = x[:, h * DH:(h + 1) * DH]
        xh = _rms(xh, w_row)
        x1, x2 = xh[:, :DH // 2], xh[:, DH // 2:]
        return jnp.concatenate([x1 * c - x2 * s, x2 * c + x1 * s], axis=-1)

    q_ref[...] = jnp.concatenate(
        [norm_rope(q, h, qw_ref[...]) for h in range(NH)], axis=-1)
    k_ref[...] = jnp.concatenate(
        [norm_rope(k, h, kw_ref[...]) for h in range(NKV)], axis=-1)


def _qkv(xn, wq, wk, wv, qw, kw, cos_t, sin_t):
    grid = (T // _BT2,)
    return pl.pallas_call(
        _qkv_body,
        grid=grid,
        in_specs=[
            pl.BlockSpec((_BT2, H), lambda i: (i, 0)),
            pl.BlockSpec((H, NH * DH), lambda i: (0, 0)),
            pl.BlockSpec((H, NKV * DH), lambda i: (0, 0)),
            pl.BlockSpec((H, NKV * DH), lambda i: (0, 0)),
            pl.BlockSpec((1, DH), lambda i: (0, 0)),
            pl.BlockSpec((1, DH), lambda i: (0, 0)),
            pl.BlockSpec((_BT2, DH // 2), lambda i: (i, 0)),
            pl.BlockSpec((_BT2, DH // 2), lambda i: (i, 0)),
        ],
        out_specs=[
            pl.BlockSpec((_BT2, NH * DH), lambda i: (i, 0)),
            pl.BlockSpec((_BT2, NKV * DH), lambda i: (i, 0)),
            pl.BlockSpec((_BT2, NKV * DH), lambda i: (i, 0)),
        ],
        out_shape=[
            jax.ShapeDtypeStruct((T, NH * DH), F32),
            jax.ShapeDtypeStruct((T, NKV * DH), F32),
            jax.ShapeDtypeStruct((T, NKV * DH), F32),
        ],
        compiler_params=pltpu.CompilerParams(
            dimension_semantics=("parallel",),
            vmem_limit_bytes=100 * 1024 * 1024,
        ),
    )(xn, wq, wk, wv, qw, kw, cos_t, sin_t)


_BQ = 256
_REP = NH // NKV


def _attn_body(q_ref, k_ref, v_ref, o_ref):
    qi = pl.program_id(1)
    q = q_ref[...]
    kf = k_ref[...]
    vf = v_ref[...]
    s = jax.lax.dot_general(q, kf, (((1,), (1,)), ((), ())),
                            preferred_element_type=F32) * (DH ** -0.5)
    row = qi * _BQ + jax.lax.broadcasted_iota(jnp.int32, (_BQ, T), 0)
    col = jax.lax.broadcasted_iota(jnp.int32, (_BQ, T), 1)
    s = jnp.where(row >= col, s, -1e30)
    m = jnp.max(s, axis=-1, keepdims=True)
    p = jnp.exp(s - m)
    l = jnp.sum(p, axis=-1, keepdims=True)
    o = jnp.dot(p, vf, preferred_element_type=F32)
    o_ref[...] = o / l


def _attn(q, k, v):
    grid = (NH, T // _BQ)
    return pl.pallas_call(
        _attn_body,
        grid=grid,
        in_specs=[
            pl.BlockSpec((_BQ, DH), lambda h, qi: (qi, h)),
            pl.BlockSpec((T, DH), lambda h, qi: (0, h // _REP)),
            pl.BlockSpec((T, DH), lambda h, qi: (0, h // _REP)),
        ],
        out_specs=pl.BlockSpec((_BQ, DH), lambda h, qi: (qi, h)),
        out_shape=jax.ShapeDtypeStruct((T, NH * DH), F32),
        compiler_params=pltpu.CompilerParams(
            dimension_semantics=("parallel", "arbitrary"),
            vmem_limit_bytes=100 * 1024 * 1024,
        ),
    )(q, k, v)


_BT4 = 256


def _first_max_mask(p):
    m = jnp.max(p, axis=-1, keepdims=True)
    iota = jax.lax.broadcasted_iota(jnp.int32, p.shape, 1)
    idx = jnp.min(jnp.where(p == m, iota, p.shape[-1]), axis=-1, keepdims=True)
    return iota == idx, m


def _oproj_body(attn_ref, wo_ref, resid_ref, wpost_ref, rw_ref,
                r2_ref, xn2_ref, gates_ref):
    x = jnp.dot(attn_ref[...], wo_ref[...], preferred_element_type=F32)
    x = x + resid_ref[...]
    r2_ref[...] = x
    xn = _rms(x, wpost_ref[...])
    xn2_ref[...] = xn
    logits = jnp.dot(xn, rw_ref[...], preferred_element_type=F32)
    lm = jnp.max(logits, axis=-1, keepdims=True)
    ex = jnp.exp(logits - lm)
    probs = ex / jnp.sum(ex, axis=-1, keepdims=True)
    first1, m1 = _first_max_mask(probs)
    p2 = jnp.where(first1, -1.0, probs)
    first2, m2 = _first_max_mask(p2)
    denom = m1 + m2
    gates_ref[...] = jnp.where(first1 | first2, probs / denom, 0.0)


def _oproj(attn_out, wo, resid, wpost, router_w):
    grid = (T // _BT4,)
    return pl.pallas_call(
        _oproj_body,
        grid=grid,
        in_specs=[
            pl.BlockSpec((_BT4, NH * DH), lambda i: (i, 0)),
            pl.BlockSpec((NH * DH, H), lambda i: (0, 0)),
            pl.BlockSpec((_BT4, H), lambda i: (i, 0)),
            pl.BlockSpec((1, H), lambda i: (0, 0)),
            pl.BlockSpec((H, E), lambda i: (0, 0)),
        ],
        out_specs=[
            pl.BlockSpec((_BT4, H), lambda i: (i, 0)),
            pl.BlockSpec((_BT4, H), lambda i: (i, 0)),
            pl.BlockSpec((_BT4, E), lambda i: (i, 0)),
        ],
        out_shape=[
            jax.ShapeDtypeStruct((T, H), F32),
            jax.ShapeDtypeStruct((T, H), F32),
            jax.ShapeDtypeStruct((T, E), F32),
        ],
        compiler_params=pltpu.CompilerParams(
            dimension_semantics=("parallel",),
            vmem_limit_bytes=100 * 1024 * 1024,
        ),
    )(attn_out, wo, resid, wpost, router_w)


_G = 128
_NT = (T * K) // _G + E
_NP = _NT * _G


def _route(gates):
    topv, topi = jax.lax.top_k(gates, K)
    flat_e = topi.reshape(-1).astype(jnp.int32)
    flat_g = topv.reshape(-1)

    oneh = (flat_e[:, None] == jnp.arange(E, dtype=jnp.int32)[None, :]
            ).astype(jnp.int32)
    csum = jnp.cumsum(oneh, axis=0)
    rank = jnp.take_along_axis(csum, flat_e[:, None], axis=1)[:, 0] - 1
    counts = csum[-1]
    padded = ((counts + _G - 1) // _G) * _G
    bounds = jnp.cumsum(padded).astype(jnp.int32)
    pstart = bounds - padded
    dest = pstart[flat_e] + rank

    tile_start = jnp.arange(_NT, dtype=jnp.int32)[:, None] * _G
    tile_expert = jnp.sum((tile_start >= bounds[None, :]).astype(jnp.int32),
                          axis=1)
    tile_expert = jnp.minimum(tile_expert, E - 1).astype(jnp.int32)

    tok = (jnp.arange(T * K, dtype=jnp.int32) // K)
    row_tok = jnp.zeros((_NP,), jnp.int32).at[dest].set(tok)
    row_gate = jnp.zeros((_NP,), F32).at[dest].set(flat_g)
    d = dest.reshape(T, K)
    return row_tok, row_gate, tile_expert, d[:, 0], d[:, 1]


def _moe_a_body(te_ref, x_ref, wg_ref, wu_ref, gb_ref, a_ref):
    x = x_ref[...]
    g = jnp.dot(x, wg_ref[0], preferred_element_type=F32)
    u = jnp.dot(x, wu_ref[0], preferred_element_type=F32)
    gate = pltpu.repeat(gb_ref[...], I // 128, axis=1)
    a_ref[...] = ((g * jax.nn.sigmoid(g)) * u * gate).astype(BF16)


def _moe_b_body(te_ref, a_ref, wd_ref, o_ref):
    o_ref[...] = jnp.dot(a_ref[...], wd_ref[0], preferred_element_type=F32)


def _moe(xn2, gates, w_gate, w_up, w_down, r2, final_w):
    row_tok, row_gate, tile_expert, d0, d1 = _route(gates)
    x_sorted = xn2.astype(BF16)[row_tok]
    gate_b = jnp.broadcast_to(row_gate[:, None], (_NP, 128))

    a_sorted = pl.pallas_call(
        _moe_a_body,
        grid_spec=pltpu.PrefetchScalarGridSpec(
            num_scalar_prefetch=1,
            grid=(_NT,),
            in_specs=[
                pl.BlockSpec((_G, H), lambda i, te: (i, 0)),
                pl.BlockSpec((1, H, I), lambda i, te: (te[i], 0, 0)),
                pl.BlockSpec((1, H, I), lambda i, te: (te[i], 0, 0)),
                pl.BlockSpec((_G, 128), lambda i, te: (i, 0)),
            ],
            out_specs=pl.BlockSpec((_G, I), lambda i, te: (i, 0)),
        ),
        out_shape=jax.ShapeDtypeStruct((_NP, I), BF16),
        compiler_params=pltpu.CompilerParams(
            dimension_semantics=("parallel",),
            vmem_limit_bytes=100 * 1024 * 1024,
        ),
    )(tile_expert, x_sorted, w_gate.astype(BF16), w_up.astype(BF16), gate_b)

    y_sorted = pl.pallas_call(
        _moe_b_body,
        grid_spec=pltpu.PrefetchScalarGridSpec(
            num_scalar_prefetch=1,
            grid=(_NT,),
            in_specs=[
                pl.BlockSpec((_G, I), lambda i, te: (i, 0)),
                pl.BlockSpec((1, I, H), lambda i, te: (te[i], 0, 0)),
            ],
            out_specs=pl.BlockSpec((_G, H), lambda i, te: (i, 0)),
        ),
        out_shape=jax.ShapeDtypeStruct((_NP, H), F32),
        compiler_params=pltpu.CompilerParams(
            dimension_semantics=("parallel",),
            vmem_limit_bytes=100 * 1024 * 1024,
        ),
    )(tile_expert, a_sorted, w_down.astype(BF16))

    y = y_sorted[d0] + y_sorted[d1]
    return _final(y, r2, final_w)


_BT6 = 256


def _final_body(y_ref, r2_ref, w_ref, out_ref):
    out_ref[...] = _rms(y_ref[...] + r2_ref[...], w_ref[...])


def _final(y, r2, w):
    grid = (T // _BT6,)
    return pl.pallas_call(
        _final_body,
        grid=grid,
        in_specs=[
            pl.BlockSpec((_BT6, H), lambda i: (i, 0)),
            pl.BlockSpec((_BT6, H), lambda i: (i, 0)),
            pl.BlockSpec((1, H), lambda i: (0, 0)),
        ],
        out_specs=pl.BlockSpec((_BT6, H), lambda i: (i, 0)),
        out_shape=jax.ShapeDtypeStruct((T, H), F32),
        compiler_params=pltpu.CompilerParams(
            dimension_semantics=("parallel",),
            vmem_limit_bytes=100 * 1024 * 1024,
        ),
    )(y, r2, w)


def kernel(input_ids, positions, hidden_states, spec_step_idx, embed_w, fc_w,
           pre_fc_emb_w, pre_fc_hid_w, in_ln_w, post_ln_w, final_norm_w,
           wq, wk, wv, wo, q_norm_w, k_norm_w, router_w, w_gate, w_up, w_down):
    ids = input_ids.astype(jnp.int32)
    row = lambda w: w.reshape(1, -1)

    resid, xn = _prefc(ids, hidden_states, fc_w, row(pre_fc_emb_w),
                       row(pre_fc_hid_w), row(in_ln_w), embed_w)

    inv = 1.0 / (THETA ** (jnp.arange(0, DH, 2, dtype=F32) / DH))
    ang = positions.astype(F32)[:, None] * inv[None, :]
    cos_t, sin_t = jnp.cos(ang), jnp.sin(ang)

    q, k, v = _qkv(xn, wq, wk, wv, row(q_norm_w), row(k_norm_w), cos_t, sin_t)
    attn_out = _attn(q, k, v)
    r2, xn2, gates = _oproj(attn_out, wo, resid, row(post_ln_w), router_w)
    return _moe(xn2, gates, w_gate, w_up, w_down, r2, row(final_norm_w))


# speedup vs baseline: 1.1610x; 1.1610x over previous
import functools

import jax
import jax.numpy as jnp
from jax.experimental import pallas as pl
from jax.experimental.pallas import tpu as pltpu

T, H, V = 2048, 2048, 32000
NH, NKV, DH = 16, 4, 128
E, K, I = 16, 2, 1024
EPS = 1e-6
THETA = 1000000.0
F32 = jnp.float32


def _rms(x, w_row):
    v = jnp.mean(x * x, axis=-1, keepdims=True)
    return x * jax.lax.rsqrt(v + EPS) * (1.0 + w_row)


BF16 = jnp.bfloat16


_BT1 = 128


def _prefc_body(ids_ref, emb_hbm, hid_ref, fc_ref, wemb_ref, whid_ref,
                win_ref, resid_ref, xn_ref, esc, sem):
    i = pl.program_id(0)
    base = i * _BT1
    copies = []
    for j in range(_BT1):
        idx = ids_ref[base + j]
        cp = pltpu.make_async_copy(
            emb_hbm.at[pl.ds(idx, 1), :], esc.at[pl.ds(j, 1), :], sem)
        cp.start()
        copies.append(cp)
    for cp in copies:
        cp.wait()
    emb_n = _rms(esc[...], wemb_ref[...])
    hid_n = _rms(hid_ref[...], whid_ref[...])
    x = (jnp.dot(emb_n, fc_ref[:H, :], preferred_element_type=F32)
         + jnp.dot(hid_n, fc_ref[H:, :], preferred_element_type=F32))
    resid_ref[...] = x
    xn_ref[...] = _rms(x, win_ref[...])


def _prefc(input_ids, hidden_states, fc_w, wemb, whid, win, embed_w):
    grid = (T // _BT1,)
    return pl.pallas_call(
        _prefc_body,
        grid_spec=pltpu.PrefetchScalarGridSpec(
            num_scalar_prefetch=1,
            grid=grid,
            in_specs=[
                pl.BlockSpec(memory_space=pl.ANY),
                pl.BlockSpec((_BT1, H), lambda i, ids: (i, 0)),
                pl.BlockSpec((2 * H, H), lambda i, ids: (0, 0)),
                pl.BlockSpec((1, H), lambda i, ids: (0, 0)),
                pl.BlockSpec((1, H), lambda i, ids: (0, 0)),
                pl.BlockSpec((1, H), lambda i, ids: (0, 0)),
            ],
            out_specs=[
                pl.BlockSpec((_BT1, H), lambda i, ids: (i, 0)),
                pl.BlockSpec((_BT1, H), lambda i, ids: (i, 0)),
            ],
            scratch_shapes=[
                pltpu.VMEM((_BT1, H), F32),
                pltpu.SemaphoreType.DMA,
            ],
        ),
        out_shape=[
            jax.ShapeDtypeStruct((T, H), F32),
            jax.ShapeDtypeStruct((T, H), F32),
        ],
        compiler_params=pltpu.CompilerParams(
            dimension_semantics=("parallel",),
            vmem_limit_bytes=100 * 1024 * 1024,
        ),
    )(input_ids, embed_w, hidden_states, fc_w, wemb, whid, win)


_BT2 = 256


def _qkv_body(xn_ref, wq_ref, wk_ref, wv_ref, qw_ref, kw_ref,
              cos_ref, sin_ref, q_ref, k_ref, v_ref):
    xn = xn_ref[...]
    q = jnp.dot(xn, wq_ref[...], preferred_element_type=F32)
    k = jnp.dot(xn, wk_ref[...], preferred_element_type=F32)
    v_ref[...] = jnp.dot(xn, wv_ref[...], preferred_element_type=F32)
    c, s = cos_ref[...], sin_ref[...]

    def norm_rope(x, h, w_row):
        xh = x[:, h * DH:(h + 1) * DH]
        xh = _rms(xh, w_row)
        x1, x2 = xh[:, :DH // 2], xh[:, DH // 2:]
        return jnp.concatenate([x1 * c - x2 * s, x2 * c + x1 * s], axis=-1)

    q_ref[...] = jnp.concatenate(
        [norm_rope(q, h, qw_ref[...]) for h in range(NH)], axis=-1)
    k_ref[...] = jnp.concatenate(
        [norm_rope(k, h, kw_ref[...]) for h in range(NKV)], axis=-1)


def _qkv(xn, wq, wk, wv, qw, kw, cos_t, sin_t):
    grid = (T // _BT2,)
    return pl.pallas_call(
        _qkv_body,
        grid=grid,
        in_specs=[
            pl.BlockSpec((_BT2, H), lambda i: (i, 0)),
            pl.BlockSpec((H, NH * DH), lambda i: (0, 0)),
            pl.BlockSpec((H, NKV * DH), lambda i: (0, 0)),
            pl.BlockSpec((H, NKV * DH), lambda i: (0, 0)),
            pl.BlockSpec((1, DH), lambda i: (0, 0)),
            pl.BlockSpec((1, DH), lambda i: (0, 0)),
            pl.BlockSpec((_BT2, DH // 2), lambda i: (i, 0)),
            pl.BlockSpec((_BT2, DH // 2), lambda i: (i, 0)),
        ],
        out_specs=[
            pl.BlockSpec((_BT2, NH * DH), lambda i: (i, 0)),
            pl.BlockSpec((_BT2, NKV * DH), lambda i: (i, 0)),
            pl.BlockSpec((_BT2, NKV * DH), lambda i: (i, 0)),
        ],
        out_shape=[
            jax.ShapeDtypeStruct((T, NH * DH), F32),
            jax.ShapeDtypeStruct((T, NKV * DH), F32),
            jax.ShapeDtypeStruct((T, NKV * DH), F32),
        ],
        compiler_params=pltpu.CompilerParams(
            dimension_semantics=("parallel",),
            vmem_limit_bytes=100 * 1024 * 1024,
        ),
    )(xn, wq, wk, wv, qw, kw, cos_t, sin_t)


_BQ = 256
_REP = NH // NKV


def _attn_body(q_ref, k_ref, v_ref, o_ref):
    qi = pl.program_id(1)
    q = q_ref[...]
    kf = k_ref[...]
    vf = v_ref[...]
    s = jax.lax.dot_general(q, kf, (((1,), (1,)), ((), ())),
                            preferred_element_type=F32) * (DH ** -0.5)
    row = qi * _BQ + jax.lax.broadcasted_iota(jnp.int32, (_BQ, T), 0)
    col = jax.lax.broadcasted_iota(jnp.int32, (_BQ, T), 1)
    s = jnp.where(row >= col, s, -1e30)
    m = jnp.max(s, axis=-1, keepdims=True)
    p = jnp.exp(s - m)
    l = jnp.sum(p, axis=-1, keepdims=True)
    o = jnp.dot(p, vf, preferred_element_type=F32)
    o_ref[...] = o / l


def _attn(q, k, v):
    grid = (NH, T // _BQ)
    return pl.pallas_call(
        _attn_body,
        grid=grid,
        in_specs=[
            pl.BlockSpec((_BQ, DH), lambda h, qi: (qi, h)),
            pl.BlockSpec((T, DH), lambda h, qi: (0, h // _REP)),
            pl.BlockSpec((T, DH), lambda h, qi: (0, h // _REP)),
        ],
        out_specs=pl.BlockSpec((_BQ, DH), lambda h, qi: (qi, h)),
        out_shape=jax.ShapeDtypeStruct((T, NH * DH), F32),
        compiler_params=pltpu.CompilerParams(
            dimension_semantics=("parallel", "arbitrary"),
            vmem_limit_bytes=100 * 1024 * 1024,
        ),
    )(q, k, v)


_BT4 = 256


def _first_max_mask(p):
    m = jnp.max(p, axis=-1, keepdims=True)
    iota = jax.lax.broadcasted_iota(jnp.int32, p.shape, 1)
    idx = jnp.min(jnp.where(p == m, iota, p.shape[-1]), axis=-1, keepdims=True)
    return iota == idx, m


def _oproj_body(attn_ref, wo_ref, resid_ref, wpost_ref, rw_ref,
                r2_ref, xn2_ref, gates_ref):
    x = jnp.dot(attn_ref[...], wo_ref[...], preferred_element_type=F32)
    x = x + resid_ref[...]
    r2_ref[...] = x
    xn = _rms(x, wpost_ref[...])
    xn2_ref[...] = xn
    logits = jnp.dot(xn, rw_ref[...], preferred_element_type=F32)
    lm = jnp.max(logits, axis=-1, keepdims=True)
    ex = jnp.exp(logits - lm)
    probs = ex / jnp.sum(ex, axis=-1, keepdims=True)
    first1, m1 = _first_max_mask(probs)
    p2 = jnp.where(first1, -1.0, probs)
    first2, m2 = _first_max_mask(p2)
    denom = m1 + m2
    gates_ref[...] = jnp.where(first1 | first2, probs / denom, 0.0)


def _oproj(attn_out, wo, resid, wpost, router_w):
    grid = (T // _BT4,)
    return pl.pallas_call(
        _oproj_body,
        grid=grid,
        in_specs=[
            pl.BlockSpec((_BT4, NH * DH), lambda i: (i, 0)),
            pl.BlockSpec((NH * DH, H), lambda i: (0, 0)),
            pl.BlockSpec((_BT4, H), lambda i: (i, 0)),
            pl.BlockSpec((1, H), lambda i: (0, 0)),
            pl.BlockSpec((H, E), lambda i: (0, 0)),
        ],
        out_specs=[
            pl.BlockSpec((_BT4, H), lambda i: (i, 0)),
            pl.BlockSpec((_BT4, H), lambda i: (i, 0)),
            pl.BlockSpec((_BT4, E), lambda i: (i, 0)),
        ],
        out_shape=[
            jax.ShapeDtypeStruct((T, H), F32),
            jax.ShapeDtypeStruct((T, H), F32),
            jax.ShapeDtypeStruct((T, E), F32),
        ],
        compiler_params=pltpu.CompilerParams(
            dimension_semantics=("parallel",),
            vmem_limit_bytes=100 * 1024 * 1024,
        ),
    )(attn_out, wo, resid, wpost, router_w)


_G = 128
_NT = (T * K) // _G + E
_NP = _NT * _G


def _route(gates):
    topv, topi = jax.lax.top_k(gates, K)
    flat_e = topi.reshape(-1).astype(jnp.int32)
    flat_g = topv.reshape(-1)

    oneh = (flat_e[:, None] == jnp.arange(E, dtype=jnp.int32)[None, :]
            ).astype(jnp.int32)
    csum = jnp.cumsum(oneh, axis=0)
    rank = jnp.take_along_axis(csum, flat_e[:, None], axis=1)[:, 0] - 1
    counts = csum[-1]
    padded = ((counts + _G - 1) // _G) * _G
    bounds = jnp.cumsum(padded).astype(jnp.int32)
    pstart = bounds - padded
    dest = pstart[flat_e] + rank

    tile_start = jnp.arange(_NT, dtype=jnp.int32)[:, None] * _G
    tile_expert = jnp.sum((tile_start >= bounds[None, :]).astype(jnp.int32),
                          axis=1)
    tile_expert = jnp.minimum(tile_expert, E - 1).astype(jnp.int32)

    tok = (jnp.arange(T * K, dtype=jnp.int32) // K)
    row_tok = jnp.zeros((_NP,), jnp.int32).at[dest].set(tok)
    row_gate = jnp.zeros((_NP,), F32).at[dest].set(flat_g)
    d = dest.reshape(T, K)
    return row_tok, row_gate, tile_expert, d[:, 0], d[:, 1]


def _moe_a_body(te_ref, x_ref, wg_ref, wu_ref, gb_ref, a_ref):
    x = x_ref[...].astype(BF16)
    g = jnp.dot(x, wg_ref[0].astype(BF16), preferred_element_type=F32)
    u = jnp.dot(x, wu_ref[0].astype(BF16), preferred_element_type=F32)
    gate = pltpu.repeat(gb_ref[...], I // 128, axis=1)
    a_ref[...] = ((g * jax.nn.sigmoid(g)) * u * gate).astype(BF16)


def _moe_b_body(te_ref, a_ref, wd_ref, o_ref):
    o_ref[...] = jnp.dot(a_ref[...], wd_ref[0].astype(BF16),
                         preferred_element_type=F32)


def _moe(xn2, gates, w_gate, w_up, w_down, r2, final_w):
    row_tok, row_gate, tile_expert, d0, d1 = _route(gates)
    x_sorted = xn2[row_tok]
    gate_b = jnp.broadcast_to(row_gate[:, None], (_NP, 128))

    a_sorted = pl.pallas_call(
        _moe_a_body,
        grid_spec=pltpu.PrefetchScalarGridSpec(
            num_scalar_prefetch=1,
            grid=(_NT,),
            in_specs=[
                pl.BlockSpec((_G, H), lambda i, te: (i, 0)),
                pl.BlockSpec((1, H, I), lambda i, te: (te[i], 0, 0)),
                pl.BlockSpec((1, H, I), lambda i, te: (te[i], 0, 0)),
                pl.BlockSpec((_G, 128), lambda i, te: (i, 0)),
            ],
            out_specs=pl.BlockSpec((_G, I), lambda i, te: (i, 0)),
        ),
        out_shape=jax.ShapeDtypeStruct((_NP, I), BF16),
        compiler_params=pltpu.CompilerParams(
            dimension_semantics=("parallel",),
            vmem_limit_bytes=100 * 1024 * 1024,
        ),
    )(tile_expert, x_sorted, w_gate, w_up, gate_b)

    y_sorted = pl.pallas_call(
        _moe_b_body,
        grid_spec=pltpu.PrefetchScalarGridSpec(
            num_scalar_prefetch=1,
            grid=(_NT,),
            in_specs=[
                pl.BlockSpec((_G, I), lambda i, te: (i, 0)),
                pl.BlockSpec((1, I, H), lambda i, te: (te[i], 0, 0)),
            ],
            out_specs=pl.BlockSpec((_G, H), lambda i, te: (i, 0)),
        ),
        out_shape=jax.ShapeDtypeStruct((_NP, H), F32),
        compiler_params=pltpu.CompilerParams(
            dimension_semantics=("parallel",),
            vmem_limit_bytes=100 * 1024 * 1024,
        ),
    )(tile_expert, a_sorted, w_down)

    y = y_sorted[d0] + y_sorted[d1]
    return _final(y, r2, final_w)


_BT6 = 256


def _final_body(y_ref, r2_ref, w_ref, out_ref):
    out_ref[...] = _rms(y_ref[...] + r2_ref[...], w_ref[...])


def _final(y, r2, w):
    grid = (T // _BT6,)
    return pl.pallas_call(
        _final_body,
        grid=grid,
        in_specs=[
            pl.BlockSpec((_BT6, H), lambda i: (i, 0)),
            pl.BlockSpec((_BT6, H), lambda i: (i, 0)),
            pl.BlockSpec((1, H), lambda i: (0, 0)),
        ],
        out_specs=pl.BlockSpec((_BT6, H), lambda i: (i, 0)),
        out_shape=jax.ShapeDtypeStruct((T, H), F32),
        compiler_params=pltpu.CompilerParams(
            dimension_semantics=("parallel",),
            vmem_limit_bytes=100 * 1024 * 1024,
        ),
    )(y, r2, w)


def kernel(input_ids, positions, hidden_states, spec_step_idx, embed_w, fc_w,
           pre_fc_emb_w, pre_fc_hid_w, in_ln_w, post_ln_w, final_norm_w,
           wq, wk, wv, wo, q_norm_w, k_norm_w, router_w, w_gate, w_up, w_down):
    ids = input_ids.astype(jnp.int32)
    row = lambda w: w.reshape(1, -1)

    resid, xn = _prefc(ids, hidden_states, fc_w, row(pre_fc_emb_w),
                       row(pre_fc_hid_w), row(in_ln_w), embed_w)

    inv = 1.0 / (THETA ** (jnp.arange(0, DH, 2, dtype=F32) / DH))
    ang = positions.astype(F32)[:, None] * inv[None, :]
    cos_t, sin_t = jnp.cos(ang), jnp.sin(ang)

    q, k, v = _qkv(xn, wq, wk, wv, row(q_norm_w), row(k_norm_w), cos_t, sin_t)
    attn_out = _attn(q, k, v)
    r2, xn2, gates = _oproj(attn_out, wo, resid, row(post_ln_w), router_w)
    return _moe(xn2, gates, w_gate, w_up, w_down, r2, row(final_norm_w))
